# Initial kernel scaffold; baseline (speedup 1.0000x reference)
import jax
import jax.numpy as jnp
from jax.experimental import pallas as pl


def kernel(x, item_emb, pos_emb):
    raise NotImplementedError("write your pallas kernel here")


# speedup vs baseline: 3.6351x; 3.6351x over previous
import functools
import math

import jax
import jax.numpy as jnp
from jax import lax
from jax.experimental import pallas as pl
from jax.experimental.pallas import tpu as pltpu
from jax.experimental.pallas import tpu_sc as plsc

_B = 1024
_S = 200
_D = 128
_ROWS = _B * _S
_NC = 2
_NS = 16
_NW = _NC * _NS
_RPW = _ROWS // _NW
_CHUNK = 128
_NCHUNK = _RPW // _CHUNK
_LANES = 16
_SCALE = math.sqrt(float(_D))


@functools.partial(
    pl.kernel,
    out_type=jax.ShapeDtypeStruct((_ROWS, _D), jnp.float32),
    mesh=plsc.VectorSubcoreMesh(core_axis_name="c", subcore_axis_name="s"),
    scratch_types=[
        pltpu.VMEM((_NCHUNK, _CHUNK), jnp.int32),
        pltpu.VMEM((_CHUNK, _D), jnp.float32),
        pltpu.VMEM((2 * _S, _D), jnp.float32),
        pltpu.SemaphoreType.DMA,
    ],
)
def _sc_embed(x_hbm, table_hbm, pos_hbm, out_hbm, idx_v, rows_v, pos2_v, sem):
    wid = lax.axis_index("s") * _NC + lax.axis_index("c")

    pltpu.sync_copy(x_hbm.at[wid], idx_v)
    pltpu.sync_copy(pos_hbm, pos2_v.at[pl.ds(0, _S)])
    pltpu.sync_copy(pos_hbm, pos2_v.at[pl.ds(_S, _S)])

    def chunk_body(k, carry):
        base = wid * _RPW + k * _CHUNK
        p0 = lax.rem(k * _CHUNK, _S)
        pltpu.async_copy(table_hbm.at[idx_v.at[k]], rows_v, sem).wait()

        def row_body(r, c2):
            for cc in range(_D // _LANES):
                sl = pl.ds(cc * _LANES, _LANES)
                rows_v[r, sl] = rows_v[r, sl] * _SCALE + pos2_v[p0 + r, sl]
            return c2

        lax.fori_loop(0, _CHUNK, row_body, 0)
        pltpu.sync_copy(rows_v, out_hbm.at[pl.ds(base, _CHUNK)])
        return carry

    lax.fori_loop(0, _NCHUNK, chunk_body, 0)


def kernel(x, item_emb, pos_emb):
    x2 = x.reshape(_NW, _NCHUNK, _CHUNK).astype(jnp.int32)
    out = _sc_embed(x2, item_emb, pos_emb)
    return out.reshape(_B, _S, _D)


# speedup vs baseline: 11.2712x; 3.1007x over previous
import functools
import math

import jax
import jax.numpy as jnp
from jax import lax
from jax.experimental import pallas as pl
from jax.experimental.pallas import tpu as pltpu
from jax.experimental.pallas import tpu_sc as plsc

_B = 1024
_S = 200
_D = 128
_ROWS = _B * _S
_NC = 2
_NS = 16
_NW = _NC * _NS
_RPW = _ROWS // _NW
_CHUNK = 128
_NCHUNK = _RPW // _CHUNK
_LANES = 16
_SCALE = math.sqrt(float(_D))


@functools.partial(
    pl.kernel,
    out_type=jax.ShapeDtypeStruct((_ROWS, _D), jnp.float32),
    mesh=plsc.VectorSubcoreMesh(core_axis_name="c", subcore_axis_name="s"),
    scratch_types=[
        pltpu.VMEM((_NCHUNK, _CHUNK), jnp.int32),
        pltpu.VMEM((2, _CHUNK, _D), jnp.float32),
        pltpu.VMEM((2 * _S, _D), jnp.float32),
        pltpu.SemaphoreType.DMA,
        pltpu.SemaphoreType.DMA,
        pltpu.SemaphoreType.DMA,
        pltpu.SemaphoreType.DMA,
    ],
)
def _sc_embed(x_hbm, table_hbm, pos_hbm, out_hbm, idx_v, rows_v, pos2_v,
              g0, g1, s0, s1):
    wid = lax.axis_index("s") * _NC + lax.axis_index("c")
    gsem = (g0, g1)
    ssem = (s0, s1)

    pltpu.sync_copy(x_hbm.at[wid], idx_v)
    pltpu.sync_copy(pos_hbm, pos2_v.at[pl.ds(0, _S)])
    pltpu.sync_copy(pos_hbm, pos2_v.at[pl.ds(_S, _S)])

    def compute(buf, k):
        p0 = (k * _CHUNK) % _S
        def row_body(r2, carry):
            for rr in range(2):
                r = r2 * 2 + rr
                for cc in range(_D // _LANES):
                    sl = pl.ds(cc * _LANES, _LANES)
                    buf[r, sl] = buf[r, sl] * _SCALE + pos2_v[p0 + r, sl]
            return carry
        lax.fori_loop(0, _CHUNK // 2, row_body, 0)

    gathers = [None, None]
    stores = [None, None]
    gathers[0] = pltpu.async_copy(table_hbm.at[idx_v.at[0]], rows_v.at[0],
                                  gsem[0])
    for k in range(_NCHUNK):
        b = k & 1
        nb = b ^ 1
        if k + 1 < _NCHUNK:
            if stores[nb] is not None:
                stores[nb].wait()
            gathers[nb] = pltpu.async_copy(
                table_hbm.at[idx_v.at[k + 1]], rows_v.at[nb], gsem[nb])
        gathers[b].wait()
        compute(rows_v.at[b], k)
        stores[b] = pltpu.async_copy(
            rows_v.at[b], out_hbm.at[pl.ds(wid * _RPW + k * _CHUNK, _CHUNK)],
            ssem[b])
    stores[0].wait()
    stores[1].wait()


def kernel(x, item_emb, pos_emb):
    x2 = x.reshape(_NW, _NCHUNK, _CHUNK).astype(jnp.int32)
    out = _sc_embed(x2, item_emb, pos_emb)
    return out.reshape(_B, _S, _D)


# speedup vs baseline: 11.3025x; 1.0028x over previous
import functools
import math

import jax
import jax.numpy as jnp
from jax import lax
from jax.experimental import pallas as pl
from jax.experimental.pallas import tpu as pltpu
from jax.experimental.pallas import tpu_sc as plsc

_B = 1024
_S = 200
_D = 128
_ROWS = _B * _S
_NC = 2
_NS = 16
_NW = _NC * _NS
_RPW = _ROWS // _NW
_CHUNK = 128
_NCHUNK = _RPW // _CHUNK
_LANES = 16
_SCALE = math.sqrt(float(_D))


@functools.partial(
    pl.kernel,
    out_type=jax.ShapeDtypeStruct((_ROWS, _D), jnp.float32),
    mesh=plsc.VectorSubcoreMesh(core_axis_name="c", subcore_axis_name="s"),
    scratch_types=[
        pltpu.VMEM((_NCHUNK, _CHUNK), jnp.int32),
        pltpu.VMEM((3, _CHUNK, _D), jnp.float32),
        pltpu.VMEM((2 * _S, _D), jnp.float32),
        pltpu.SemaphoreType.DMA,
        pltpu.SemaphoreType.DMA,
        pltpu.SemaphoreType.DMA,
        pltpu.SemaphoreType.DMA,
        pltpu.SemaphoreType.DMA,
        pltpu.SemaphoreType.DMA,
    ],
)
def _sc_embed(x_hbm, table_hbm, pos_hbm, out_hbm, idx_v, rows_v, pos2_v,
              g0, g1, g2, s0, s1, s2):
    wid = lax.axis_index("s") * _NC + lax.axis_index("c")
    gsem = (g0, g1, g2)
    ssem = (s0, s1, s2)
    nbuf = 3

    pltpu.sync_copy(x_hbm.at[wid], idx_v)
    pltpu.sync_copy(pos_hbm, pos2_v.at[pl.ds(0, _S)])
    pltpu.sync_copy(pos_hbm, pos2_v.at[pl.ds(_S, _S)])

    def compute(buf, k):
        p0 = (k * _CHUNK) % _S
        def row_body(r2, carry):
            for rr in range(2):
                r = r2 * 2 + rr
                for cc in range(_D // _LANES):
                    sl = pl.ds(cc * _LANES, _LANES)
                    buf[r, sl] = buf[r, sl] * _SCALE + pos2_v[p0 + r, sl]
            return carry
        lax.fori_loop(0, _CHUNK // 2, row_body, 0)

    gathers = [None] * nbuf
    stores = [None] * nbuf
    for k in range(nbuf - 1):
        gathers[k] = pltpu.async_copy(table_hbm.at[idx_v.at[k]],
                                      rows_v.at[k], gsem[k])
    for k in range(_NCHUNK):
        b = k % nbuf
        ka = k + nbuf - 1
        if ka < _NCHUNK:
            ba = ka % nbuf
            if stores[ba] is not None:
                stores[ba].wait()
            gathers[ba] = pltpu.async_copy(
                table_hbm.at[idx_v.at[ka]], rows_v.at[ba], gsem[ba])
        gathers[b].wait()
        compute(rows_v.at[b], k)
        stores[b] = pltpu.async_copy(
            rows_v.at[b], out_hbm.at[pl.ds(wid * _RPW + k * _CHUNK, _CHUNK)],
            ssem[b])
    for b in range(nbuf):
        if stores[b] is not None:
            stores[b].wait()


def kernel(x, item_emb, pos_emb):
    x2 = x.reshape(_NW, _NCHUNK, _CHUNK).astype(jnp.int32)
    out = _sc_embed(x2, item_emb, pos_emb)
    return out.reshape(_B, _S, _D)
